# baseline (speedup 1.0000x reference)
import functools
import math

import jax
import jax.numpy as jnp
from jax import lax
from jax.experimental import pallas as pl
from jax.experimental.pallas import tpu as pltpu
from jax.experimental.pallas import tpu_sc as plsc

HEAD_DIM = 128
NUM_HEADS = 32
NUM_GROUPS = 8
HEADS_PER_GROUP = NUM_HEADS // NUM_GROUPS
NUM_BLOCKS = 64
BLOCK = 64
NSEL = 16
BATCH = 4
SEQ = 4096
NW = 32
ROWS_PER_W = NSEL * BLOCK
CHUNK = 128
NCHUNK = ROWS_PER_W // CHUNK
NJOBS = NCHUNK
NBUF = 7
DEPTH = 5


def _score_topk_body(q_ref, ck_ref, out_ref, bid_ref):
    diags = []
    for b in range(BATCH):
        q = q_ref[b, :, 0, :]
        ck = ck_ref[b].reshape(NUM_HEADS * NUM_BLOCKS, HEAD_DIM)
        s = lax.dot_general(q, ck, (((1,), (1,)), ((), ())))
        s3 = s.reshape(NUM_HEADS, NUM_HEADS, NUM_BLOCKS)
        hi = lax.broadcasted_iota(jnp.int32, s3.shape, 0)
        hj = lax.broadcasted_iota(jnp.int32, s3.shape, 1)
        diags.append(jnp.sum(jnp.where(hi == hj, s3, 0.0), axis=1))
    s2 = jnp.concatenate(diags, axis=0) / math.sqrt(HEAD_DIM)
    m = jnp.max(s2, axis=-1, keepdims=True)
    e = jnp.exp(s2 - m)
    p = e / jnp.sum(e, axis=-1, keepdims=True)
    p3d = p.reshape(NW, HEADS_PER_GROUP, NUM_BLOCKS)
    pg = (((p3d[:, 0, :] + p3d[:, 1, :]) + p3d[:, 2, :]) + p3d[:, 3, :]) / 4.0
    ii = lax.broadcasted_iota(jnp.int32, (NW, NUM_BLOCKS), 1)
    gflat = lax.broadcasted_iota(jnp.int32, (NW, NUM_BLOCKS), 0)
    base = gflat * SEQ
    cur = pg
    for r in range(NSEL):
        mx = jnp.max(cur, axis=-1, keepdims=True)
        am = jnp.min(jnp.where(cur == mx, ii, NUM_BLOCKS),
                     axis=-1, keepdims=True)
        rows = base + am * BLOCK + ii
        out_ref[:, r // 2, pl.ds((r % 2) * BLOCK, BLOCK)] = rows
        bid_ref[:, pl.ds(r, 1)] = gflat[:, :1] * NUM_BLOCKS + am
        cur = jnp.where(ii == am, -1.0, cur)


def _score_topk(query, compressed_keys, interpret=False):
    return pl.pallas_call(
        _score_topk_body,
        in_specs=[
            pl.BlockSpec((BATCH, NUM_HEADS, 1, HEAD_DIM),
                         lambda: (0, 0, 0, 0)),
            pl.BlockSpec((BATCH, NUM_HEADS, NUM_BLOCKS, HEAD_DIM),
                         lambda: (0, 0, 0, 0)),
        ],
        out_specs=[pl.BlockSpec((NW, NCHUNK, CHUNK), lambda: (0, 0, 0)),
                   pl.BlockSpec((NW, NSEL), lambda: (0, 0))],
        out_shape=[jax.ShapeDtypeStruct((NW, NCHUNK, CHUNK), jnp.int32),
                   jax.ShapeDtypeStruct((NW, NSEL), jnp.int32)],
        interpret=interpret,
    )(query, compressed_keys)


def _sc_gather_body(kt, gidx, outk, idx_v, bufs, gsems, ssems):
    cid = lax.axis_index("c")
    sid = lax.axis_index("s")
    wid = sid * 2 + cid
    pltpu.sync_copy(gidx.at[wid], idx_v)
    obase = wid * ROWS_PER_W

    def fire_gather(j):
        return pltpu.async_copy(kt.at[idx_v.at[j]],
                                bufs[j % NBUF], gsems[j % NBUF])

    def fire_scatter(j):
        dst = outk.at[pl.ds(obase + j * CHUNK, CHUNK)]
        return pltpu.async_copy(bufs[j % NBUF], dst, ssems[j % NBUF])

    hg = {}
    hs = {}
    for j in range(DEPTH):
        hg[j] = fire_gather(j)
    for j in range(NJOBS):
        hg[j].wait()
        hs[j] = fire_scatter(j)
        nxt = j + DEPTH
        if nxt < NJOBS:
            prev = nxt - NBUF
            if prev >= 0:
                hs[prev].wait()
            hg[nxt] = fire_gather(nxt)
    for j in range(max(0, NJOBS - NBUF), NJOBS):
        hs[j].wait()


@functools.cache
def _sc_gather():
    def body(kt, gidx, outk, idx_v, *rest):
        bufs = rest[:NBUF]
        gsems = rest[NBUF:2 * NBUF]
        ssems = rest[2 * NBUF:]
        _sc_gather_body(kt, gidx, outk, idx_v, bufs, gsems, ssems)

    return pl.kernel(
        body,
        out_type=jax.ShapeDtypeStruct((NW * ROWS_PER_W, HEAD_DIM),
                                      jnp.float32),
        mesh=plsc.VectorSubcoreMesh(core_axis_name="c", subcore_axis_name="s"),
        scratch_types=(
            [pltpu.VMEM((NCHUNK, CHUNK), jnp.int32)]
            + [pltpu.VMEM((CHUNK, HEAD_DIM), jnp.float32)] * NBUF
            + [pltpu.SemaphoreType.DMA] * (2 * NBUF)
        ),
    )


def _tc_gather_body(bid_ref, v_ref, out_ref):
    out_ref[...] = v_ref[...]


def _tc_gather(bids_flat, vt3):
    return pl.pallas_call(
        _tc_gather_body,
        grid_spec=pltpu.PrefetchScalarGridSpec(
            num_scalar_prefetch=1,
            grid=(NW * NSEL,),
            in_specs=[pl.BlockSpec((1, BLOCK, HEAD_DIM),
                                   lambda j, bid: (bid[j], 0, 0))],
            out_specs=pl.BlockSpec((1, BLOCK, HEAD_DIM),
                                   lambda j, bid: (j, 0, 0)),
        ),
        out_shape=jax.ShapeDtypeStruct((NW * NSEL, BLOCK, HEAD_DIM),
                                       jnp.float32),
    )(bids_flat, vt3)


def kernel(query, compressed_keys, keys, values):
    gidx, bids = _score_topk(query, compressed_keys)
    kt = keys.reshape(NW * SEQ, HEAD_DIM)
    vt3 = values.reshape(NW * NUM_BLOCKS, BLOCK, HEAD_DIM)
    outk = _sc_gather()(kt, gidx)
    outv = _tc_gather(bids.reshape(NW * NSEL), vt3)
    return (outk.reshape(BATCH, NUM_GROUPS, ROWS_PER_W, HEAD_DIM),
            outv.reshape(BATCH, NUM_GROUPS, ROWS_PER_W, HEAD_DIM))


# speedup vs baseline: 5.6946x; 5.6946x over previous
import functools
import math

import jax
import jax.numpy as jnp
from jax import lax
from jax.experimental import pallas as pl
from jax.experimental.pallas import tpu as pltpu
from jax.experimental.pallas import tpu_sc as plsc

HEAD_DIM = 128
NUM_HEADS = 32
NUM_GROUPS = 8
HEADS_PER_GROUP = NUM_HEADS // NUM_GROUPS
NUM_BLOCKS = 64
BLOCK = 64
NSEL = 16
BATCH = 4
SEQ = 4096
NW = 32
ROWS_PER_W = NSEL * BLOCK
CHUNK = 64
NCHUNK = ROWS_PER_W // CHUNK
NJOBS = 2 * NCHUNK
NBUF = 12
DEPTH = 8


def _score_topk_body(q_ref, ck_ref, out_ref):
    diags = []
    for b in range(BATCH):
        q = q_ref[b, :, 0, :]
        ck = ck_ref[b].reshape(NUM_HEADS * NUM_BLOCKS, HEAD_DIM)
        s = lax.dot_general(q, ck, (((1,), (1,)), ((), ())))
        s3 = s.reshape(NUM_HEADS, NUM_HEADS, NUM_BLOCKS)
        hi = lax.broadcasted_iota(jnp.int32, s3.shape, 0)
        hj = lax.broadcasted_iota(jnp.int32, s3.shape, 1)
        diags.append(jnp.sum(jnp.where(hi == hj, s3, 0.0), axis=1))
    s2 = jnp.concatenate(diags, axis=0) / math.sqrt(HEAD_DIM)
    m = jnp.max(s2, axis=-1, keepdims=True)
    e = jnp.exp(s2 - m)
    p = e / jnp.sum(e, axis=-1, keepdims=True)
    p3d = p.reshape(NW, HEADS_PER_GROUP, NUM_BLOCKS)
    pg = (((p3d[:, 0, :] + p3d[:, 1, :]) + p3d[:, 2, :]) + p3d[:, 3, :]) / 4.0
    ii = lax.broadcasted_iota(jnp.int32, (NW, NUM_BLOCKS), 1)
    gflat = lax.broadcasted_iota(jnp.int32, (NW, NUM_BLOCKS), 0)
    base = gflat * SEQ
    cur = pg
    for r in range(NSEL):
        mx = jnp.max(cur, axis=-1, keepdims=True)
        am = jnp.min(jnp.where(cur == mx, ii, NUM_BLOCKS),
                     axis=-1, keepdims=True)
        rows = base + am * BLOCK + ii
        out_ref[:, (r * BLOCK) // CHUNK,
                pl.ds((r * BLOCK) % CHUNK, BLOCK)] = rows
        cur = jnp.where(ii == am, -1.0, cur)


def _score_topk(query, compressed_keys, interpret=False):
    return pl.pallas_call(
        _score_topk_body,
        in_specs=[
            pl.BlockSpec((BATCH, NUM_HEADS, 1, HEAD_DIM),
                         lambda: (0, 0, 0, 0)),
            pl.BlockSpec((BATCH, NUM_HEADS, NUM_BLOCKS, HEAD_DIM),
                         lambda: (0, 0, 0, 0)),
        ],
        out_specs=pl.BlockSpec((NW, NCHUNK, CHUNK), lambda: (0, 0, 0)),
        out_shape=jax.ShapeDtypeStruct((NW, NCHUNK, CHUNK), jnp.int32),
        interpret=interpret,
    )(query, compressed_keys)


def _sc_gather_body(kt, vt, gidx, outk, outv, idx_v, bufs, gsems, ssems):
    cid = lax.axis_index("c")
    sid = lax.axis_index("s")
    wid = sid * 2 + cid
    pltpu.sync_copy(gidx.at[wid], idx_v)
    obase = wid * ROWS_PER_W
    tabs = (kt, vt)
    outs = (outk, outv)

    def fire_gather(j):
        path, chunk = j % 2, j // 2
        return pltpu.async_copy(tabs[path].at[idx_v.at[chunk]],
                                bufs[j % NBUF], gsems[j % NBUF])

    def fire_scatter(j):
        path, chunk = j % 2, j // 2
        dst = outs[path].at[pl.ds(obase + chunk * CHUNK, CHUNK)]
        return pltpu.async_copy(bufs[j % NBUF], dst, ssems[j % NBUF])

    hg = {}
    hs = {}
    for j in range(DEPTH):
        hg[j] = fire_gather(j)
    for j in range(NJOBS):
        hg[j].wait()
        hs[j] = fire_scatter(j)
        nxt = j + DEPTH
        if nxt < NJOBS:
            prev = nxt - NBUF
            if prev >= 0:
                hs[prev].wait()
            hg[nxt] = fire_gather(nxt)
    for j in range(NJOBS - NBUF, NJOBS):
        hs[j].wait()


@functools.cache
def _sc_gather():
    def body(kt, vt, gidx, outk, outv, idx_v, *rest):
        bufs = rest[:NBUF]
        gsems = rest[NBUF:2 * NBUF]
        ssems = rest[2 * NBUF:]
        _sc_gather_body(kt, vt, gidx, outk, outv, idx_v, bufs, gsems, ssems)

    return pl.kernel(
        body,
        out_type=(
            jax.ShapeDtypeStruct((NW * ROWS_PER_W, HEAD_DIM), jnp.float32),
            jax.ShapeDtypeStruct((NW * ROWS_PER_W, HEAD_DIM), jnp.float32),
        ),
        mesh=plsc.VectorSubcoreMesh(core_axis_name="c", subcore_axis_name="s"),
        scratch_types=(
            [pltpu.VMEM((NCHUNK, CHUNK), jnp.int32)]
            + [pltpu.VMEM((CHUNK, HEAD_DIM), jnp.float32)] * NBUF
            + [pltpu.SemaphoreType.DMA] * (2 * NBUF)
        ),
    )


def kernel(query, compressed_keys, keys, values):
    gidx = _score_topk(query, compressed_keys)
    kt = keys.reshape(NW * SEQ, HEAD_DIM)
    vt = values.reshape(NW * SEQ, HEAD_DIM)
    outk, outv = _sc_gather()(kt, vt, gidx)
    return (outk.reshape(BATCH, NUM_GROUPS, ROWS_PER_W, HEAD_DIM),
            outv.reshape(BATCH, NUM_GROUPS, ROWS_PER_W, HEAD_DIM))


# speedup vs baseline: 5.6952x; 1.0001x over previous
import functools
import math

import jax
import jax.numpy as jnp
from jax import lax
from jax.experimental import pallas as pl
from jax.experimental.pallas import tpu as pltpu
from jax.experimental.pallas import tpu_sc as plsc

HEAD_DIM = 128
NUM_HEADS = 32
NUM_GROUPS = 8
HEADS_PER_GROUP = NUM_HEADS // NUM_GROUPS
NUM_BLOCKS = 64
BLOCK = 64
NSEL = 16
BATCH = 4
SEQ = 4096
NW = 32
ROWS_PER_W = NSEL * BLOCK
CHUNK = 128
NCHUNK = ROWS_PER_W // CHUNK
NJOBS = 2 * NCHUNK
NBUF = 7
DEPTH = 5


def _score_topk_body(q_ref, ck_ref, out_ref, pg_ref):
    b = pl.program_id(0)
    q = q_ref[0, :, 0, :]
    ck = ck_ref[0].reshape(NUM_HEADS * NUM_BLOCKS, HEAD_DIM)
    s = lax.dot_general(q, ck, (((1,), (1,)), ((), ())))
    s3 = s.reshape(NUM_HEADS, NUM_HEADS, NUM_BLOCKS)
    hi = lax.broadcasted_iota(jnp.int32, s3.shape, 0)
    hj = lax.broadcasted_iota(jnp.int32, s3.shape, 1)
    s2 = jnp.sum(jnp.where(hi == hj, s3, 0.0), axis=1) / math.sqrt(HEAD_DIM)
    m = jnp.max(s2, axis=-1, keepdims=True)
    e = jnp.exp(s2 - m)
    p = e / jnp.sum(e, axis=-1, keepdims=True)
    p3d = p.reshape(NUM_GROUPS, HEADS_PER_GROUP, NUM_BLOCKS)
    pg_ref[pl.ds(b * NUM_GROUPS, NUM_GROUPS), :] = (
        ((p3d[:, 0, :] + p3d[:, 1, :]) + p3d[:, 2, :]) + p3d[:, 3, :]) / 4.0

    @pl.when(b == BATCH - 1)
    def _():
        ii = lax.broadcasted_iota(jnp.int32, (NW, NUM_BLOCKS), 1)
        gflat = lax.broadcasted_iota(jnp.int32, (NW, NUM_BLOCKS), 0)
        base = gflat * SEQ
        cur = pg_ref[...]
        for r in range(NSEL):
            mx = jnp.max(cur, axis=-1, keepdims=True)
            am = jnp.min(jnp.where(cur == mx, ii, NUM_BLOCKS),
                         axis=-1, keepdims=True)
            rows = base + am * BLOCK + ii
            out_ref[:, r // 2, pl.ds((r % 2) * BLOCK, BLOCK)] = rows
            cur = jnp.where(ii == am, -1.0, cur)


def _score_topk(query, compressed_keys, interpret=False):
    return pl.pallas_call(
        _score_topk_body,
        grid=(BATCH,),
        in_specs=[
            pl.BlockSpec((1, NUM_HEADS, 1, HEAD_DIM),
                         lambda b: (b, 0, 0, 0)),
            pl.BlockSpec((1, NUM_HEADS, NUM_BLOCKS, HEAD_DIM),
                         lambda b: (b, 0, 0, 0)),
        ],
        out_specs=pl.BlockSpec((NW, NCHUNK, CHUNK), lambda b: (0, 0, 0)),
        out_shape=jax.ShapeDtypeStruct((NW, NCHUNK, CHUNK), jnp.int32),
        scratch_shapes=[pltpu.VMEM((NW, NUM_BLOCKS), jnp.float32)],
        interpret=interpret,
    )(query, compressed_keys)


def _sc_gather_body(kt, vt, gidx, outk, outv, idx_v, bufs, gsems, ssems):
    cid = lax.axis_index("c")
    sid = lax.axis_index("s")
    wid = sid * 2 + cid
    pltpu.sync_copy(gidx.at[wid], idx_v)
    obase = wid * ROWS_PER_W
    tabs = (kt, vt)
    outs = (outk, outv)

    def fire_gather(j):
        path, chunk = j % 2, j // 2
        return pltpu.async_copy(tabs[path].at[idx_v.at[chunk]],
                                bufs[j % NBUF], gsems[j % NBUF])

    def fire_scatter(j):
        path, chunk = j % 2, j // 2
        dst = outs[path].at[pl.ds(obase + chunk * CHUNK, CHUNK)]
        return pltpu.async_copy(bufs[j % NBUF], dst, ssems[j % NBUF])

    hg = {}
    hs = {}
    for j in range(DEPTH):
        hg[j] = fire_gather(j)
    for j in range(NJOBS):
        hg[j].wait()
        hs[j] = fire_scatter(j)
        nxt = j + DEPTH
        if nxt < NJOBS:
            prev = nxt - NBUF
            if prev >= 0:
                hs[prev].wait()
            hg[nxt] = fire_gather(nxt)
    for j in range(NJOBS - NBUF, NJOBS):
        hs[j].wait()


@functools.cache
def _sc_gather():
    def body(kt, vt, gidx, outk, outv, idx_v, *rest):
        bufs = rest[:NBUF]
        gsems = rest[NBUF:2 * NBUF]
        ssems = rest[2 * NBUF:]
        _sc_gather_body(kt, vt, gidx, outk, outv, idx_v, bufs, gsems, ssems)

    return pl.kernel(
        body,
        out_type=(
            jax.ShapeDtypeStruct((NW * ROWS_PER_W, HEAD_DIM), jnp.float32),
            jax.ShapeDtypeStruct((NW * ROWS_PER_W, HEAD_DIM), jnp.float32),
        ),
        mesh=plsc.VectorSubcoreMesh(core_axis_name="c", subcore_axis_name="s"),
        scratch_types=(
            [pltpu.VMEM((NCHUNK, CHUNK), jnp.int32)]
            + [pltpu.VMEM((CHUNK, HEAD_DIM), jnp.float32)] * NBUF
            + [pltpu.SemaphoreType.DMA] * (2 * NBUF)
        ),
    )


def kernel(query, compressed_keys, keys, values):
    gidx = _score_topk(query, compressed_keys)
    kt = keys.reshape(NW * SEQ, HEAD_DIM)
    vt = values.reshape(NW * SEQ, HEAD_DIM)
    outk, outv = _sc_gather()(kt, vt, gidx)
    return (outk.reshape(BATCH, NUM_GROUPS, ROWS_PER_W, HEAD_DIM),
            outv.reshape(BATCH, NUM_GROUPS, ROWS_PER_W, HEAD_DIM))


# speedup vs baseline: 5.7777x; 1.0145x over previous
import functools
import math

import jax
import jax.numpy as jnp
from jax import lax
from jax.experimental import pallas as pl
from jax.experimental.pallas import tpu as pltpu
from jax.experimental.pallas import tpu_sc as plsc

HEAD_DIM = 128
NUM_HEADS = 32
NUM_GROUPS = 8
HEADS_PER_GROUP = NUM_HEADS // NUM_GROUPS
NUM_BLOCKS = 64
BLOCK = 64
NSEL = 16
BATCH = 4
SEQ = 4096
NW = 32
ROWS_PER_W = NSEL * BLOCK
CHUNK = 128
NCHUNK = ROWS_PER_W // CHUNK
NJOBS = 2 * NCHUNK
NBUF = 7
DEPTH = 6


def _score_topk_body(q_ref, ck_ref, out_ref):
    diags = []
    for b in range(BATCH):
        q = q_ref[b, :, 0, :]
        ck = ck_ref[b].reshape(NUM_HEADS * NUM_BLOCKS, HEAD_DIM)
        s = lax.dot_general(q, ck, (((1,), (1,)), ((), ())))
        s3 = s.reshape(NUM_HEADS, NUM_HEADS, NUM_BLOCKS)
        hi = lax.broadcasted_iota(jnp.int32, s3.shape, 0)
        hj = lax.broadcasted_iota(jnp.int32, s3.shape, 1)
        diags.append(jnp.sum(jnp.where(hi == hj, s3, 0.0), axis=1))
    s2 = jnp.concatenate(diags, axis=0) / math.sqrt(HEAD_DIM)
    m = jnp.max(s2, axis=-1, keepdims=True)
    e = jnp.exp(s2 - m)
    p = e / jnp.sum(e, axis=-1, keepdims=True)
    p3d = p.reshape(NW, HEADS_PER_GROUP, NUM_BLOCKS)
    pg = (((p3d[:, 0, :] + p3d[:, 1, :]) + p3d[:, 2, :]) + p3d[:, 3, :]) / 4.0
    ii = lax.broadcasted_iota(jnp.int32, (NW, NUM_BLOCKS), 1)
    gflat = lax.broadcasted_iota(jnp.int32, (NW, NUM_BLOCKS), 0)
    base = gflat * SEQ
    cur = pg
    for r in range(NSEL):
        mx = jnp.max(cur, axis=-1, keepdims=True)
        am = jnp.min(jnp.where(cur == mx, ii, NUM_BLOCKS),
                     axis=-1, keepdims=True)
        rows = base + am * BLOCK + ii
        out_ref[:, r // 2, pl.ds((r % 2) * BLOCK, BLOCK)] = rows
        cur = jnp.where(ii == am, -1.0, cur)


def _score_topk(query, compressed_keys, interpret=False):
    return pl.pallas_call(
        _score_topk_body,
        in_specs=[
            pl.BlockSpec((BATCH, NUM_HEADS, 1, HEAD_DIM),
                         lambda: (0, 0, 0, 0)),
            pl.BlockSpec((BATCH, NUM_HEADS, NUM_BLOCKS, HEAD_DIM),
                         lambda: (0, 0, 0, 0)),
        ],
        out_specs=pl.BlockSpec((NW, NCHUNK, CHUNK), lambda: (0, 0, 0)),
        out_shape=jax.ShapeDtypeStruct((NW, NCHUNK, CHUNK), jnp.int32),
        interpret=interpret,
    )(query, compressed_keys)


def _sc_gather_body(kt, vt, gidx, outk, outv, idx_v, bufs, gsems, ssems):
    cid = lax.axis_index("c")
    sid = lax.axis_index("s")
    wid = sid * 2 + cid
    pltpu.sync_copy(gidx.at[wid], idx_v)
    obase = wid * ROWS_PER_W
    tabs = (kt, vt)
    outs = (outk, outv)

    def fire_gather(j):
        path, chunk = j % 2, j // 2
        return pltpu.async_copy(tabs[path].at[idx_v.at[chunk]],
                                bufs[j % NBUF], gsems[j % NBUF])

    def fire_scatter(j):
        path, chunk = j % 2, j // 2
        dst = outs[path].at[pl.ds(obase + chunk * CHUNK, CHUNK)]
        return pltpu.async_copy(bufs[j % NBUF], dst, ssems[j % NBUF])

    hg = {}
    hs = {}
    for j in range(DEPTH):
        hg[j] = fire_gather(j)
    for j in range(NJOBS):
        hg[j].wait()
        hs[j] = fire_scatter(j)
        nxt = j + DEPTH
        if nxt < NJOBS:
            prev = nxt - NBUF
            if prev >= 0:
                hs[prev].wait()
            hg[nxt] = fire_gather(nxt)
    for j in range(NJOBS - NBUF, NJOBS):
        hs[j].wait()


@functools.cache
def _sc_gather():
    def body(kt, vt, gidx, outk, outv, idx_v, *rest):
        bufs = rest[:NBUF]
        gsems = rest[NBUF:2 * NBUF]
        ssems = rest[2 * NBUF:]
        _sc_gather_body(kt, vt, gidx, outk, outv, idx_v, bufs, gsems, ssems)

    return pl.kernel(
        body,
        out_type=(
            jax.ShapeDtypeStruct((NW * ROWS_PER_W, HEAD_DIM), jnp.float32),
            jax.ShapeDtypeStruct((NW * ROWS_PER_W, HEAD_DIM), jnp.float32),
        ),
        mesh=plsc.VectorSubcoreMesh(core_axis_name="c", subcore_axis_name="s"),
        scratch_types=(
            [pltpu.VMEM((NCHUNK, CHUNK), jnp.int32)]
            + [pltpu.VMEM((CHUNK, HEAD_DIM), jnp.float32)] * NBUF
            + [pltpu.SemaphoreType.DMA] * (2 * NBUF)
        ),
    )


def kernel(query, compressed_keys, keys, values):
    gidx = _score_topk(query, compressed_keys)
    kt = keys.reshape(NW * SEQ, HEAD_DIM)
    vt = values.reshape(NW * SEQ, HEAD_DIM)
    outk, outv = _sc_gather()(kt, vt, gidx)
    return (outk.reshape(BATCH, NUM_GROUPS, ROWS_PER_W, HEAD_DIM),
            outv.reshape(BATCH, NUM_GROUPS, ROWS_PER_W, HEAD_DIM))
